# baseline (speedup 1.0000x reference)
import functools

import jax
import jax.numpy as jnp
from jax.experimental import pallas as pl
from jax.experimental.pallas import tpu as pltpu

_VMEM_LIMIT_BYTES = 64 * 1024 * 1024
_NCORES = 2


def _layer1_kernel(adj_ref, b_ref, w_ref, adjb_ref, out_ref, bb_ref, wb_ref):
    @pl.when(pl.program_id(1) == 0)
    def _():
        bb_ref[...] = b_ref[...].astype(jnp.bfloat16)
        wb_ref[...] = w_ref[...].astype(jnp.bfloat16)

    adj_b = adj_ref[...].astype(jnp.bfloat16)
    adjb_ref[...] = adj_b
    h = jnp.dot(adj_b, bb_ref[...], preferred_element_type=jnp.float32)
    h = jnp.maximum(h, 0.0).astype(jnp.bfloat16)
    out_ref[...] = jnp.dot(
        h, wb_ref[...], preferred_element_type=jnp.float32
    ).astype(out_ref.dtype)


def _layer2_kernel(adj_ref, b_ref, w_ref, out_ref, wb_ref):
    @pl.when(pl.program_id(1) == 0)
    def _():
        wb_ref[...] = w_ref[...].astype(jnp.bfloat16)

    h = jnp.dot(adj_ref[...], b_ref[...], preferred_element_type=jnp.float32)
    h = jnp.maximum(h, 0.0).astype(jnp.bfloat16)
    out_ref[...] = jnp.dot(
        h, wb_ref[...], preferred_element_type=jnp.float32
    ).astype(out_ref.dtype)


def _final_kernel(adj_ref, b_ref, out_ref):
    out_ref[...] = jnp.dot(
        adj_ref[...], b_ref[...], preferred_element_type=jnp.float32
    ).astype(out_ref.dtype)


def _grid_specs(m, tm):
    s = m // tm // _NCORES
    grid = (_NCORES, s)
    row = pl.BlockSpec((tm, None), lambda i, j, _s=s: (i * _s + j, 0))
    return grid, row, s


def _layer1(adj_f32, b, w, *, tm):
    m, k = adj_f32.shape
    h = b.shape[1]
    c = w.shape[1]
    s = m // tm // _NCORES
    idx = lambda i, j: (i * s + j, 0)
    const = lambda i, j: (0, 0)
    return pl.pallas_call(
        _layer1_kernel,
        out_shape=(
            jax.ShapeDtypeStruct((m, k), jnp.bfloat16),
            jax.ShapeDtypeStruct((m, c), jnp.bfloat16),
        ),
        grid=(_NCORES, s),
        in_specs=[
            pl.BlockSpec((tm, k), idx),
            pl.BlockSpec((k, h), const),
            pl.BlockSpec((h, c), const),
        ],
        out_specs=(
            pl.BlockSpec((tm, k), idx),
            pl.BlockSpec((tm, c), idx),
        ),
        scratch_shapes=[
            pltpu.VMEM((k, h), jnp.bfloat16),
            pltpu.VMEM((h, c), jnp.bfloat16),
        ],
        compiler_params=pltpu.CompilerParams(
            dimension_semantics=("parallel", "arbitrary"),
            vmem_limit_bytes=_VMEM_LIMIT_BYTES,
        ),
    )(adj_f32, b, w)


def _layer2(adj_b, b, w, *, tm):
    m, k = adj_b.shape
    h = b.shape[1]
    c = w.shape[1]
    s = m // tm // _NCORES
    idx = lambda i, j: (i * s + j, 0)
    const = lambda i, j: (0, 0)
    return pl.pallas_call(
        _layer2_kernel,
        out_shape=jax.ShapeDtypeStruct((m, c), jnp.bfloat16),
        grid=(_NCORES, s),
        in_specs=[
            pl.BlockSpec((tm, k), idx),
            pl.BlockSpec((k, h), const),
            pl.BlockSpec((h, c), const),
        ],
        out_specs=pl.BlockSpec((tm, c), idx),
        scratch_shapes=[pltpu.VMEM((h, c), jnp.bfloat16)],
        compiler_params=pltpu.CompilerParams(
            dimension_semantics=("parallel", "arbitrary"),
            vmem_limit_bytes=_VMEM_LIMIT_BYTES,
        ),
    )(adj_b, b, w)


def _final_matmul(adj_b, b, *, tm):
    m, k = adj_b.shape
    c = b.shape[1]
    s = m // tm // _NCORES
    idx = lambda i, j: (i * s + j, 0)
    const = lambda i, j: (0, 0)
    return pl.pallas_call(
        _final_kernel,
        out_shape=jax.ShapeDtypeStruct((m, c), jnp.float32),
        grid=(_NCORES, s),
        in_specs=[
            pl.BlockSpec((tm, k), idx),
            pl.BlockSpec((k, c), const),
        ],
        out_specs=pl.BlockSpec((tm, c), idx),
        compiler_params=pltpu.CompilerParams(
            dimension_semantics=("parallel", "arbitrary"),
            vmem_limit_bytes=_VMEM_LIMIT_BYTES,
        ),
    )(adj_b, b)


def kernel(W1, W2, Wmid0, x, adj):
    del x
    n = adj.shape[0]
    assert n % 512 == 0, adj.shape
    tm = 256

    adj_b, pre1 = _layer1(adj, W1, Wmid0, tm=tm)
    pre2 = _layer2(adj_b, pre1, W2, tm=tm)
    return _final_matmul(adj_b, pre2, tm=tm)


# speedup vs baseline: 1.2089x; 1.2089x over previous
import functools

import jax
import jax.numpy as jnp
from jax.experimental import pallas as pl
from jax.experimental.pallas import tpu as pltpu

_VMEM_LIMIT_BYTES = 64 * 1024 * 1024
_NCORES = 2


def _layer1_kernel(adj_ref, b_ref, w_ref, adjb_ref, out_ref, bb_ref, wb_ref):
    @pl.when(pl.program_id(1) == 0)
    def _():
        bb_ref[...] = b_ref[...].astype(jnp.bfloat16)
        wb_ref[...] = w_ref[...].astype(jnp.bfloat16)

    adj_b = adj_ref[...].astype(jnp.bfloat16)
    adjb_ref[...] = adj_b
    h = jnp.dot(adj_b, bb_ref[...], preferred_element_type=jnp.float32)
    h = jnp.maximum(h, 0.0).astype(jnp.bfloat16)
    out_ref[...] = jnp.dot(
        h, wb_ref[...], preferred_element_type=jnp.float32
    ).astype(out_ref.dtype)


def _layer2_kernel(adj_ref, b_ref, w_ref, out_ref, wb_ref):
    @pl.when(pl.program_id(1) == 0)
    def _():
        wb_ref[...] = w_ref[...].astype(jnp.bfloat16)

    h = jnp.dot(adj_ref[...], b_ref[...], preferred_element_type=jnp.float32)
    h = jnp.maximum(h, 0.0).astype(jnp.bfloat16)
    out_ref[...] = jnp.dot(
        h, wb_ref[...], preferred_element_type=jnp.float32
    ).astype(out_ref.dtype)


def _final_kernel(adj_ref, b_ref, out_ref):
    out_ref[...] = jnp.dot(
        adj_ref[...], b_ref[...], preferred_element_type=jnp.float32
    ).astype(out_ref.dtype)


def _grid_specs(m, tm):
    s = m // tm // _NCORES
    grid = (_NCORES, s)
    row = pl.BlockSpec((tm, None), lambda i, j, _s=s: (i * _s + j, 0))
    return grid, row, s


def _layer1(adj_f32, b, w, *, tm):
    m, k = adj_f32.shape
    h = b.shape[1]
    c = w.shape[1]
    s = m // tm // _NCORES
    idx = lambda i, j: (i * s + j, 0)
    const = lambda i, j: (0, 0)
    return pl.pallas_call(
        _layer1_kernel,
        out_shape=(
            jax.ShapeDtypeStruct((m, k), jnp.bfloat16),
            jax.ShapeDtypeStruct((m, c), jnp.bfloat16),
        ),
        grid=(_NCORES, s),
        in_specs=[
            pl.BlockSpec((tm, k), idx),
            pl.BlockSpec((k, h), const),
            pl.BlockSpec((h, c), const),
        ],
        out_specs=(
            pl.BlockSpec((tm, k), idx),
            pl.BlockSpec((tm, c), idx),
        ),
        scratch_shapes=[
            pltpu.VMEM((k, h), jnp.bfloat16),
            pltpu.VMEM((h, c), jnp.bfloat16),
        ],
        compiler_params=pltpu.CompilerParams(
            dimension_semantics=("parallel", "arbitrary"),
            vmem_limit_bytes=_VMEM_LIMIT_BYTES,
        ),
    )(adj_f32, b, w)


def _layer2(adj_b, b, w, *, tm):
    m, k = adj_b.shape
    h = b.shape[1]
    c = w.shape[1]
    s = m // tm // _NCORES
    idx = lambda i, j: (i * s + j, 0)
    const = lambda i, j: (0, 0)
    return pl.pallas_call(
        _layer2_kernel,
        out_shape=jax.ShapeDtypeStruct((m, c), jnp.bfloat16),
        grid=(_NCORES, s),
        in_specs=[
            pl.BlockSpec((tm, k), idx),
            pl.BlockSpec((k, h), const),
            pl.BlockSpec((h, c), const),
        ],
        out_specs=pl.BlockSpec((tm, c), idx),
        scratch_shapes=[pltpu.VMEM((h, c), jnp.bfloat16)],
        compiler_params=pltpu.CompilerParams(
            dimension_semantics=("parallel", "arbitrary"),
            vmem_limit_bytes=_VMEM_LIMIT_BYTES,
        ),
    )(adj_b, b, w)


def _final_matmul(adj_b, b, *, tm):
    m, k = adj_b.shape
    c = b.shape[1]
    s = m // tm // _NCORES
    idx = lambda i, j: (i * s + j, 0)
    const = lambda i, j: (0, 0)
    return pl.pallas_call(
        _final_kernel,
        out_shape=jax.ShapeDtypeStruct((m, c), jnp.float32),
        grid=(_NCORES, s),
        in_specs=[
            pl.BlockSpec((tm, k), idx),
            pl.BlockSpec((k, c), const),
        ],
        out_specs=pl.BlockSpec((tm, c), idx),
        compiler_params=pltpu.CompilerParams(
            dimension_semantics=("parallel", "arbitrary"),
            vmem_limit_bytes=_VMEM_LIMIT_BYTES,
        ),
    )(adj_b, b)


def kernel(W1, W2, Wmid0, x, adj):
    del x
    n = adj.shape[0]
    assert n % 512 == 0, adj.shape
    tm = 512

    adj_b, pre1 = _layer1(adj, W1, Wmid0, tm=tm)
    pre2 = _layer2(adj_b, pre1, W2, tm=tm)
    return _final_matmul(adj_b, pre2, tm=tm)


# speedup vs baseline: 1.2792x; 1.0582x over previous
import functools

import jax
import jax.numpy as jnp
from jax.experimental import pallas as pl
from jax.experimental.pallas import tpu as pltpu

_VMEM_LIMIT_BYTES = 64 * 1024 * 1024
_NCORES = 2


def _layer1_kernel(adj_ref, b_ref, w_ref, adjb_ref, out_ref, bb_ref, wb_ref):
    @pl.when(pl.program_id(1) == 0)
    def _():
        bb_ref[...] = b_ref[...].astype(jnp.bfloat16)
        wb_ref[...] = w_ref[...].astype(jnp.bfloat16)

    adj_b = adj_ref[...].astype(jnp.bfloat16)
    adjb_ref[...] = adj_b
    h = jnp.dot(adj_b, bb_ref[...], preferred_element_type=jnp.float32)
    h = jnp.maximum(h, 0.0).astype(jnp.bfloat16)
    out_ref[...] = jnp.dot(
        h, wb_ref[...], preferred_element_type=jnp.float32
    ).astype(out_ref.dtype)


def _layer2_kernel(adj_ref, b_ref, w_ref, out_ref, wb_ref):
    @pl.when(pl.program_id(1) == 0)
    def _():
        wb_ref[...] = w_ref[...].astype(jnp.bfloat16)

    h = jnp.dot(adj_ref[...], b_ref[...], preferred_element_type=jnp.float32)
    h = jnp.maximum(h, 0.0).astype(jnp.bfloat16)
    out_ref[...] = jnp.dot(
        h, wb_ref[...], preferred_element_type=jnp.float32
    ).astype(out_ref.dtype)


def _final_kernel(adj_ref, b_ref, out_ref):
    out_ref[...] = jnp.dot(
        adj_ref[...], b_ref[...], preferred_element_type=jnp.float32
    ).astype(out_ref.dtype)


def _grid_specs(m, tm):
    s = m // tm // _NCORES
    grid = (_NCORES, s)
    row = pl.BlockSpec((tm, None), lambda i, j, _s=s: (i * _s + j, 0))
    return grid, row, s


def _layer1(adj_f32, b, w, *, tm):
    m, k = adj_f32.shape
    h = b.shape[1]
    c = w.shape[1]
    s = m // tm // _NCORES
    idx = lambda i, j: (i * s + j, 0)
    const = lambda i, j: (0, 0)
    return pl.pallas_call(
        _layer1_kernel,
        out_shape=(
            jax.ShapeDtypeStruct((m, k), jnp.bfloat16),
            jax.ShapeDtypeStruct((m, c), jnp.bfloat16),
        ),
        grid=(_NCORES, s),
        in_specs=[
            pl.BlockSpec((tm, k), idx),
            pl.BlockSpec((k, h), const),
            pl.BlockSpec((h, c), const),
        ],
        out_specs=(
            pl.BlockSpec((tm, k), idx),
            pl.BlockSpec((tm, c), idx),
        ),
        scratch_shapes=[
            pltpu.VMEM((k, h), jnp.bfloat16),
            pltpu.VMEM((h, c), jnp.bfloat16),
        ],
        compiler_params=pltpu.CompilerParams(
            dimension_semantics=("parallel", "arbitrary"),
            vmem_limit_bytes=_VMEM_LIMIT_BYTES,
        ),
    )(adj_f32, b, w)


def _layer2(adj_b, b, w, *, tm):
    m, k = adj_b.shape
    h = b.shape[1]
    c = w.shape[1]
    s = m // tm // _NCORES
    idx = lambda i, j: (i * s + j, 0)
    const = lambda i, j: (0, 0)
    return pl.pallas_call(
        _layer2_kernel,
        out_shape=jax.ShapeDtypeStruct((m, c), jnp.bfloat16),
        grid=(_NCORES, s),
        in_specs=[
            pl.BlockSpec((tm, k), idx),
            pl.BlockSpec((k, h), const),
            pl.BlockSpec((h, c), const),
        ],
        out_specs=pl.BlockSpec((tm, c), idx),
        scratch_shapes=[pltpu.VMEM((h, c), jnp.bfloat16)],
        compiler_params=pltpu.CompilerParams(
            dimension_semantics=("parallel", "arbitrary"),
            vmem_limit_bytes=_VMEM_LIMIT_BYTES,
        ),
    )(adj_b, b, w)


def _final_matmul(adj_b, b, *, tm):
    m, k = adj_b.shape
    c = b.shape[1]
    s = m // tm // _NCORES
    idx = lambda i, j: (i * s + j, 0)
    const = lambda i, j: (0, 0)
    return pl.pallas_call(
        _final_kernel,
        out_shape=jax.ShapeDtypeStruct((m, c), jnp.float32),
        grid=(_NCORES, s),
        in_specs=[
            pl.BlockSpec((tm, k), idx),
            pl.BlockSpec((k, c), const),
        ],
        out_specs=pl.BlockSpec((tm, c), idx),
        compiler_params=pltpu.CompilerParams(
            dimension_semantics=("parallel", "arbitrary"),
            vmem_limit_bytes=_VMEM_LIMIT_BYTES,
        ),
    )(adj_b, b)


def kernel(W1, W2, Wmid0, x, adj):
    del x
    n = adj.shape[0]
    assert n % 512 == 0, adj.shape
    tm = 1024

    adj_b, pre1 = _layer1(adj, W1, Wmid0, tm=tm)
    pre2 = _layer2(adj_b, pre1, W2, tm=tm)
    return _final_matmul(adj_b, pre2, tm=tm)
